# baseline (speedup 1.0000x reference)
import functools

import jax
import jax.numpy as jnp
from jax import lax
from jax.experimental import pallas as pl
from jax.experimental.pallas import tpu as pltpu
from jax.experimental.pallas import tpu_sc as plsc

NC = 2
NS = 16
NW = NC * NS
EPS = 1e-5
_BN_INV = float(1.0 / (1.0 + EPS) ** 0.5)


def _make_agg(npad: int, d: int, e: int, chunk: int, nbuf: int = 5):
  group = chunk * nbuf
  epw = e // NW
  groups = epw // group
  assert epw % group == 0 and chunk % 8 == 0
  rows = npad // NS
  mesh = plsc.VectorSubcoreMesh(core_axis_name="c", subcore_axis_name="s")

  @functools.partial(
      pl.kernel,
      out_type=jax.ShapeDtypeStruct((2 * npad, d), jnp.float32),
      mesh=mesh,
      scratch_types=[
          pltpu.VMEM_SHARED((npad, d), jnp.float32),
          pltpu.VMEM((group,), jnp.int32),
          pltpu.VMEM((group,), jnp.int32),
          [pltpu.VMEM((chunk, d), jnp.float32) for _ in range(nbuf)],
          [pltpu.SemaphoreType.DMA for _ in range(nbuf)],
      ],
  )
  def agg(table, src, dst, zeros, out, acc, srcv, dstv, stages, sems):
    cid = lax.axis_index("c")
    sid = lax.axis_index("s")
    wid = sid * NC + cid
    r0 = sid * rows
    pltpu.sync_copy(zeros.at[pl.ds(r0, rows)], acc.at[pl.ds(r0, rows)])
    plsc.subcore_barrier()
    base = wid * epw

    def body(j, carry):
      off = base + j * group
      pltpu.sync_copy(src.at[pl.ds(off, group)], srcv)
      pltpu.sync_copy(dst.at[pl.ds(off, group)], dstv)
      copies = []
      for b in range(nbuf):
        cp = pltpu.make_async_copy(
            table.at[srcv.at[pl.ds(b * chunk, chunk)]], stages[b], sems[b])
        cp.start()
        copies.append(cp)
      for b in range(nbuf):
        copies[b].wait()
        pltpu.sync_copy(stages[b],
                        acc.at[dstv.at[pl.ds(b * chunk, chunk)]], add=True)
      return carry

    lax.fori_loop(0, groups, body, 0)
    plsc.subcore_barrier()
    pltpu.sync_copy(acc.at[pl.ds(r0, rows)], out.at[pl.ds(cid * npad + r0, rows)])

  return agg


def _mm1_body(x_ref, w_ref, o_ref):
  n = x_ref.shape[0]
  o_ref[:n, :] = jnp.dot(x_ref[...], w_ref[...],
                         preferred_element_type=jnp.float32)
  o_ref[n:, :] = jnp.zeros((o_ref.shape[0] - n, o_ref.shape[1]), jnp.float32)


def _fuse_body(p_ref, b_ref, g_ref, be_ref, w_ref, o_ref):
  npad = o_ref.shape[0]
  a = p_ref[:npad, :] + p_ref[npad:, :]
  z = (a + b_ref[...]) * (g_ref[...] * _BN_INV) + be_ref[...]
  z = jnp.maximum(z, 0.0)
  o_ref[...] = jnp.dot(z, w_ref[...], preferred_element_type=jnp.float32)


def _final_body(p_ref, b_ref, o_ref):
  n, c = o_ref.shape
  npad = p_ref.shape[0] // 2
  z = p_ref[:n, :c] + p_ref[npad:npad + n, :c] + b_ref[...]
  m = jnp.max(z, axis=-1, keepdims=True)
  s = jnp.log(jnp.sum(jnp.exp(z - m), axis=-1, keepdims=True))
  o_ref[...] = z - m - s


def _tc(body, out_shape, *args):
  return pl.pallas_call(body, out_shape=out_shape)(*args)


def kernel(x, edge_index, W1, b1, g1, be1, W2, b2, g2, be2, W3, b3):
  n, ddim = x.shape
  e = edge_index.shape[1]
  h = W1.shape[1]
  c = W3.shape[1]
  npad = ((n + 8 * NS - 1) // (8 * NS)) * (8 * NS)
  cpad = 128

  src = edge_index[0]
  dst = edge_index[1]
  zeros_h = jnp.zeros((npad, h), jnp.float32)
  zeros_c = jnp.zeros((npad, cpad), jnp.float32)
  w3p = jnp.zeros((h, cpad), jnp.float32).at[:, :c].set(W3)

  b1r, g1r, be1r = b1[None, :], g1[None, :], be1[None, :]
  b2r, g2r, be2r = b2[None, :], g2[None, :], be2[None, :]
  b3r = b3[None, :]

  agg_h = _make_agg(npad, h, e, 40)
  agg_c = _make_agg(npad, cpad, e, 40)

  f32 = jnp.float32
  h1 = _tc(_mm1_body, jax.ShapeDtypeStruct((npad, h), f32), x, W1)
  p1 = agg_h(h1, src, dst, zeros_h)
  h2 = _tc(_fuse_body, jax.ShapeDtypeStruct((npad, h), f32),
           p1, b1r, g1r, be1r, W2)
  p2 = agg_h(h2, src, dst, zeros_h)
  h3 = _tc(_fuse_body, jax.ShapeDtypeStruct((npad, cpad), f32),
           p2, b2r, g2r, be2r, w3p)
  p3 = agg_c(h3, src, dst, zeros_c)
  out = _tc(_final_body, jax.ShapeDtypeStruct((n, c), f32), p3, b3r)
  return out


# speedup vs baseline: 1.0427x; 1.0427x over previous
import functools

import jax
import jax.numpy as jnp
from jax import lax
from jax.experimental import pallas as pl
from jax.experimental.pallas import tpu as pltpu
from jax.experimental.pallas import tpu_sc as plsc

NC = 2
NS = 16
NW = NC * NS
EPS = 1e-5
_BN_INV = float(1.0 / (1.0 + EPS) ** 0.5)


def _make_agg(npad: int, d: int, e: int, chunk: int, nbuf: int = 5):
  group = chunk * nbuf
  epw = e // NW
  groups = epw // group
  assert epw % group == 0 and chunk % 8 == 0
  rows = npad // NS
  mesh = plsc.VectorSubcoreMesh(core_axis_name="c", subcore_axis_name="s")

  @functools.partial(
      pl.kernel,
      out_type=jax.ShapeDtypeStruct((2 * npad, d), jnp.float32),
      mesh=mesh,
      scratch_types=[
          pltpu.VMEM_SHARED((npad, d), jnp.float32),
          pltpu.VMEM((group,), jnp.int32),
          pltpu.VMEM((group,), jnp.int32),
          [pltpu.VMEM((chunk, d), jnp.float32) for _ in range(nbuf)],
          [pltpu.SemaphoreType.DMA for _ in range(nbuf)],
          [pltpu.SemaphoreType.DMA for _ in range(nbuf)],
      ],
  )
  def agg(table, src, dst, zeros, out, acc, srcv, dstv, stages, sems, ssems):
    cid = lax.axis_index("c")
    sid = lax.axis_index("s")
    wid = sid * NC + cid
    r0 = sid * rows
    pltpu.sync_copy(zeros.at[pl.ds(r0, rows)], acc.at[pl.ds(r0, rows)])
    plsc.subcore_barrier()
    base = wid * epw

    def body(j, carry):
      off = base + j * group
      pltpu.sync_copy(src.at[pl.ds(off, group)], srcv)
      pltpu.sync_copy(dst.at[pl.ds(off, group)], dstv)
      copies = []
      for b in range(nbuf):
        cp = pltpu.make_async_copy(
            table.at[srcv.at[pl.ds(b * chunk, chunk)]], stages[b], sems[b])
        cp.start()
        copies.append(cp)
      scopies = []
      for b in range(nbuf):
        copies[b].wait()
        scopies.append(pltpu.async_copy(
            stages[b], acc.at[dstv.at[pl.ds(b * chunk, chunk)]], ssems[b],
            add=True))
      for b in range(nbuf):
        scopies[b].wait()
      return carry

    lax.fori_loop(0, groups, body, 0)
    plsc.subcore_barrier()
    pltpu.sync_copy(acc.at[pl.ds(r0, rows)], out.at[pl.ds(cid * npad + r0, rows)])

  return agg


def _mm1_body(x_ref, w_ref, o_ref):
  n = x_ref.shape[0]
  o_ref[:n, :] = jnp.dot(x_ref[...], w_ref[...],
                         preferred_element_type=jnp.float32)
  o_ref[n:, :] = jnp.zeros((o_ref.shape[0] - n, o_ref.shape[1]), jnp.float32)


def _fuse_body(p_ref, b_ref, g_ref, be_ref, w_ref, o_ref):
  npad = o_ref.shape[0]
  a = p_ref[:npad, :] + p_ref[npad:, :]
  z = (a + b_ref[...]) * (g_ref[...] * _BN_INV) + be_ref[...]
  z = jnp.maximum(z, 0.0)
  o_ref[...] = jnp.dot(z, w_ref[...], preferred_element_type=jnp.float32)


def _final_body(p_ref, b_ref, o_ref):
  n, c = o_ref.shape
  npad = p_ref.shape[0] // 2
  z = p_ref[:n, :c] + p_ref[npad:npad + n, :c] + b_ref[...]
  m = jnp.max(z, axis=-1, keepdims=True)
  s = jnp.log(jnp.sum(jnp.exp(z - m), axis=-1, keepdims=True))
  o_ref[...] = z - m - s


def _tc(body, out_shape, *args):
  return pl.pallas_call(body, out_shape=out_shape)(*args)


def kernel(x, edge_index, W1, b1, g1, be1, W2, b2, g2, be2, W3, b3):
  n, ddim = x.shape
  e = edge_index.shape[1]
  h = W1.shape[1]
  c = W3.shape[1]
  npad = ((n + 8 * NS - 1) // (8 * NS)) * (8 * NS)
  cpad = 128

  src = edge_index[0]
  dst = edge_index[1]
  zeros_h = jnp.zeros((npad, h), jnp.float32)
  zeros_c = jnp.zeros((npad, cpad), jnp.float32)
  w3p = jnp.zeros((h, cpad), jnp.float32).at[:, :c].set(W3)

  b1r, g1r, be1r = b1[None, :], g1[None, :], be1[None, :]
  b2r, g2r, be2r = b2[None, :], g2[None, :], be2[None, :]
  b3r = b3[None, :]

  agg_h = _make_agg(npad, h, e, 40)
  agg_c = _make_agg(npad, cpad, e, 40)

  f32 = jnp.float32
  h1 = _tc(_mm1_body, jax.ShapeDtypeStruct((npad, h), f32), x, W1)
  p1 = agg_h(h1, src, dst, zeros_h)
  h2 = _tc(_fuse_body, jax.ShapeDtypeStruct((npad, h), f32),
           p1, b1r, g1r, be1r, W2)
  p2 = agg_h(h2, src, dst, zeros_h)
  h3 = _tc(_fuse_body, jax.ShapeDtypeStruct((npad, cpad), f32),
           p2, b2r, g2r, be2r, w3p)
  p3 = agg_c(h3, src, dst, zeros_c)
  out = _tc(_final_body, jax.ShapeDtypeStruct((n, c), f32), p3, b3r)
  return out
